# baseline (speedup 1.0000x reference)
import functools

import jax
import jax.numpy as jnp
from jax import lax
from jax.experimental import pallas as pl
from jax.experimental.pallas import tpu as pltpu
from jax.experimental.pallas import tpu_sc as plsc

_NC = 2
_NS = 16
_NW = _NC * _NS

_DIM = 32
_BATCH = 16384
_HIST = 200
_B = _BATCH * _HIST

_CHUNK = 512
_SPB = _BATCH // _CHUNK
_NUNIT = _HIST * _SPB
_PER_W = _NUNIT // _NW
_TPC = _CHUNK // 128


_V = 1000000
_CB = 512
_NBLK = 1954


def _make_conv():
  mesh = plsc.VectorSubcoreMesh(
      core_axis_name="c", subcore_axis_name="s",
      num_cores=_NC, num_subcores=_NS)

  @functools.partial(
      pl.kernel,
      out_type=jax.ShapeDtypeStruct((_V * _DIM,), jnp.float32),
      mesh=mesh,
      scratch_types=(
          [pltpu.VMEM((_DIM, _CB), jnp.float32) for _ in range(2)]
          + [pltpu.VMEM((_CB * _DIM,), jnp.float32) for _ in range(2)]
          + [pltpu.SemaphoreType.DMA] * 2
      ),
      compiler_params=pltpu.CompilerParams(
          use_tc_tiling_on_sc=False, needs_layout_passes=False),
  )
  def conv_kernel(tt_hbm, out_hbm, in0, in1, ob0, ob1, sem_i, sem_o):
    in_bufs = (in0, in1)
    out_bufs = (ob0, ob1)
    wid = lax.axis_index("s") * _NC + lax.axis_index("c")
    cnt = jnp.where(wid < 2, 62, 61)
    start = jnp.where(wid < 2, wid * 62, 61 * wid + 2)
    lane = lax.iota(jnp.int32, 16)

    def v0_of(i):
      return jnp.minimum((start + i) * _CB, _V - _CB)

    def issue_in(i, p):
      pltpu.async_copy(
          tt_hbm.at[:, pl.ds(v0_of(i), _CB)], in_bufs[p], sem_i)

    def wait_in():
      pltpu.make_async_copy(
          tt_hbm.at[:, pl.ds(0, _CB)], in_bufs[0], sem_i).wait()

    def issue_out(i, p):
      pltpu.async_copy(
          out_bufs[p],
          out_hbm.at[pl.ds(v0_of(i) * _DIM, _CB * _DIM)], sem_o)

    def wait_out():
      pltpu.make_async_copy(
          out_bufs[0], out_hbm.at[pl.ds(0, _CB * _DIM)], sem_o).wait()

    def transpose_block(p):
      tb = in_bufs[p]
      ob = out_bufs[p]

      @plsc.parallel_loop(0, _DIM, 1, unroll=2)
      def _(d):
        rowv = (lane + d) & (_DIM - 1)
        destv = lane * _DIM + rowv
        for g in range(_CB // 16):
          v = plsc.load_gather(tb, [rowv, lane + 16 * g])
          plsc.store_scatter(ob, [destv + g * (16 * _DIM)], v)

    issue_in(0, 0)

    def blk_body(i, carry):
      for p in range(2):
        j = 2 * i + p

        @pl.when(j < cnt)
        def _():
          wait_in()

          @pl.when(j + 1 < cnt)
          def _():
            issue_in(j + 1, 1 - p)

          @pl.when(j >= 2)
          def _():
            wait_out()

          transpose_block(p)
          issue_out(j, p)
      return carry

    lax.fori_loop(0, 31, blk_body, 0)
    wait_out()
    wait_out()

  return conv_kernel


_CONV = _make_conv()


def _make_kernel():
  mesh = plsc.VectorSubcoreMesh(
      core_axis_name="c", subcore_axis_name="s",
      num_cores=_NC, num_subcores=_NS)

  @functools.partial(
      pl.kernel,
      out_type=jax.ShapeDtypeStruct((_B * _DIM,), jnp.float32),
      mesh=mesh,
      scratch_types=(
          [pltpu.VMEM((_CHUNK,), jnp.int32) for _ in range(2)]
          + [pltpu.VMEM((_CHUNK, _DIM), jnp.float32) for _ in range(2)]
          + [pltpu.VMEM((_CHUNK * _DIM,), jnp.float32) for _ in range(2)]
          + [pltpu.SemaphoreType.DMA] * 3
      ),
      compiler_params=pltpu.CompilerParams(
          use_tc_tiling_on_sc=False, needs_layout_passes=False),
  )
  def gather_kernel(xt_hbm, table_hbm, out_hbm, idx0, idx1, rows0, rows1,
                    tr0, tr1, sem_i, sem_g, sem_s):
    idx_bufs = (idx0, idx1)
    row_bufs = (rows0, rows1)
    tr_bufs = (tr0, tr1)

    wid = lax.axis_index("s") * _NC + lax.axis_index("c")
    base_u = wid * _PER_W
    lane = lax.iota(jnp.int32, 16)

    def unit_hs(i):
      u = base_u + i
      return u // _SPB, u % _SPB

    def issue_idx(i, p):
      h, s = unit_hs(i)
      pltpu.async_copy(
          xt_hbm.at[pl.ds(h * _BATCH + s * _CHUNK, _CHUNK)],
          idx_bufs[p], sem_i)

    def issue_gather(p):
      pltpu.async_copy(table_hbm.at[idx_bufs[p]], row_bufs[p], sem_g)

    def issue_store(i, p):
      h, s = unit_hs(i)
      for dt in range(_DIM // 8):
        off = ((h * (_DIM // 8) + dt) * (_BATCH // 128) + s * _TPC) * 1024
        pltpu.async_copy(
            tr_bufs[p].at[pl.ds(dt * (_TPC * 1024), _TPC * 1024)],
            out_hbm.at[pl.ds(off, _TPC * 1024)], sem_s)

    def wait_idx():
      pltpu.make_async_copy(
          xt_hbm.at[pl.ds(0, _CHUNK)], idx_bufs[0], sem_i).wait()

    def wait_gather():
      pltpu.make_async_copy(
          table_hbm.at[idx_bufs[0]], row_bufs[0], sem_g).wait()

    def wait_store():
      for dt in range(_DIM // 8):
        pltpu.make_async_copy(
            tr_bufs[0].at[pl.ds(0, _TPC * 1024)],
            out_hbm.at[pl.ds(0, _TPC * 1024)], sem_s).wait()

    def transpose_unit(p):
      rows = row_bufs[p]
      tr = tr_bufs[p]

      @plsc.parallel_loop(0, _DIM, 1, unroll=2)
      def _(d):
        colv = (lane + d) & (_DIM - 1)
        destv = ((colv >> 3) * (_TPC * 1024) + (colv & 7) * 128) + lane
        for bl in range(_TPC):
          for g in range(8):
            rowv = lane + (bl * 128 + 16 * g)
            v = plsc.load_gather(rows, [rowv, colv])
            plsc.store_scatter(tr, [destv + (bl * 1024 + 16 * g)], v)

    issue_idx(0, 0)
    issue_idx(1, 1)
    wait_idx()
    issue_gather(0)

    def round_body(r, carry):
      for p in range(2):
        i = 2 * r + p
        @pl.when(i + 1 < _PER_W)
        def _():
          wait_idx()
          issue_gather(1 - p)

        wait_gather()

        @pl.when(i >= 2)
        def _():
          wait_store()

        transpose_unit(p)
        issue_store(i, p)

        @pl.when(i + 2 < _PER_W)
        def _():
          issue_idx(i + 2, p)
      return carry

    lax.fori_loop(0, _PER_W // 2, round_body, 0)
    wait_store()
    wait_store()

  return gather_kernel


_GATHER = _make_kernel()


def kernel(x, table):
  xt = jnp.transpose(x).reshape(-1).astype(jnp.int32)
  tconv = _CONV(jnp.transpose(table))
  u = _GATHER(xt, tconv.reshape(_V, _DIM))
  u5 = u.reshape(_HIST, _DIM // 8, _BATCH // 128, 8, 128)
  v = jnp.transpose(u5, (0, 1, 3, 2, 4)).reshape(_HIST, _DIM, _BATCH)
  return jnp.transpose(v, (2, 0, 1))


# speedup vs baseline: 3.6962x; 3.6962x over previous
import functools

import jax
import jax.numpy as jnp
from jax import lax
from jax.experimental import pallas as pl
from jax.experimental.pallas import tpu as pltpu
from jax.experimental.pallas import tpu_sc as plsc

_NC = 2
_NS = 16
_NW = _NC * _NS

_DIM = 32
_BATCH = 16384
_HIST = 200
_B = _BATCH * _HIST

_CHUNK = 512
_SPB = _BATCH // _CHUNK
_NUNIT = _HIST * _SPB
_PER_W = _NUNIT // _NW
_TPC = _CHUNK // 128


_V = 1000000
_VP = 1000064
_TC7 = _VP // 128
_CB = 512
_NBLK = _VP // _CB


def _make_conv():
  mesh = plsc.VectorSubcoreMesh(
      core_axis_name="c", subcore_axis_name="s",
      num_cores=_NC, num_subcores=_NS)

  @functools.partial(
      pl.kernel,
      out_type=jax.ShapeDtypeStruct((_VP * _DIM,), jnp.float32),
      mesh=mesh,
      scratch_types=(
          [pltpu.VMEM((_CB * _DIM,), jnp.float32) for _ in range(2)]
          + [pltpu.VMEM((_CB * _DIM,), jnp.float32) for _ in range(2)]
          + [pltpu.SemaphoreType.DMA] * 2
      ),
      compiler_params=pltpu.CompilerParams(
          use_tc_tiling_on_sc=False, needs_layout_passes=False),
  )
  def conv_kernel(tt_hbm, out_hbm, in0, in1, ob0, ob1, sem_i, sem_o):
    in_bufs = (in0, in1)
    out_bufs = (ob0, ob1)
    wid = lax.axis_index("s") * _NC + lax.axis_index("c")
    cnt = jnp.where(wid < 2, 62, 61)
    start = jnp.where(wid < 2, wid * 62, 61 * wid + 2)
    lane = lax.iota(jnp.int32, 16)

    def c0_of(i):
      return jnp.minimum((start + i) * 4, _TC7 - 4)

    def issue_in(i, p):
      c0 = c0_of(i)
      for r in range(_DIM // 8):
        pltpu.async_copy(
            tt_hbm.at[pl.ds((r * _TC7 + c0) * 1024, 4 * 1024)],
            in_bufs[p].at[pl.ds(r * 4096, 4096)], sem_i)

    def wait_in():
      for r in range(_DIM // 8):
        pltpu.make_async_copy(
            tt_hbm.at[pl.ds(0, 4 * 1024)],
            in_bufs[0].at[pl.ds(0, 4096)], sem_i).wait()

    def issue_out(i, p):
      pltpu.async_copy(
          out_bufs[p],
          out_hbm.at[pl.ds(c0_of(i) * 128 * _DIM, _CB * _DIM)], sem_o)

    def wait_out():
      pltpu.make_async_copy(
          out_bufs[0], out_hbm.at[pl.ds(0, _CB * _DIM)], sem_o).wait()

    def transpose_block(p):
      tb = in_bufs[p]
      ob = out_bufs[p]

      @plsc.parallel_loop(0, _DIM, 1, unroll=2)
      def _(d):
        colv = (lane + d) & (_DIM - 1)
        srcv = (colv >> 3) * 4096 + (colv & 7) * 128 + lane
        destv = lane * _DIM + colv
        for cl in range(4):
          for g in range(8):
            v = plsc.load_gather(tb, [srcv + (cl * 1024 + 16 * g)])
            plsc.store_scatter(
                ob, [destv + (cl * 128 + 16 * g) * _DIM], v)

    issue_in(0, 0)

    def blk_body(i, carry):
      for p in range(2):
        j = 2 * i + p

        @pl.when(j < cnt)
        def _():
          wait_in()

          @pl.when(j + 1 < cnt)
          def _():
            issue_in(j + 1, 1 - p)

          @pl.when(j >= 2)
          def _():
            wait_out()

          transpose_block(p)
          issue_out(j, p)
      return carry

    lax.fori_loop(0, 31, blk_body, 0)
    wait_out()
    wait_out()

  return conv_kernel


_CONV = _make_conv()


def _make_kernel():
  mesh = plsc.VectorSubcoreMesh(
      core_axis_name="c", subcore_axis_name="s",
      num_cores=_NC, num_subcores=_NS)

  @functools.partial(
      pl.kernel,
      out_type=jax.ShapeDtypeStruct((_B * _DIM,), jnp.float32),
      mesh=mesh,
      scratch_types=(
          [pltpu.VMEM((_CHUNK,), jnp.int32) for _ in range(2)]
          + [pltpu.VMEM((_CHUNK, _DIM), jnp.float32) for _ in range(2)]
          + [pltpu.VMEM((_CHUNK * _DIM,), jnp.float32) for _ in range(2)]
          + [pltpu.SemaphoreType.DMA] * 3
      ),
      compiler_params=pltpu.CompilerParams(
          use_tc_tiling_on_sc=False, needs_layout_passes=False),
  )
  def gather_kernel(xt_hbm, table_hbm, out_hbm, idx0, idx1, rows0, rows1,
                    tr0, tr1, sem_i, sem_g, sem_s):
    idx_bufs = (idx0, idx1)
    row_bufs = (rows0, rows1)
    tr_bufs = (tr0, tr1)

    wid = lax.axis_index("s") * _NC + lax.axis_index("c")
    base_u = wid * _PER_W
    lane = lax.iota(jnp.int32, 16)

    def unit_hs(i):
      u = base_u + i
      return u // _SPB, u % _SPB

    def issue_idx(i, p):
      h, s = unit_hs(i)
      pltpu.async_copy(
          xt_hbm.at[pl.ds(h * _BATCH + s * _CHUNK, _CHUNK)],
          idx_bufs[p], sem_i)

    def issue_gather(p):
      pltpu.async_copy(table_hbm.at[idx_bufs[p]], row_bufs[p], sem_g)

    def issue_store(i, p):
      h, s = unit_hs(i)
      for dt in range(_DIM // 8):
        off = ((h * (_DIM // 8) + dt) * (_BATCH // 128) + s * _TPC) * 1024
        pltpu.async_copy(
            tr_bufs[p].at[pl.ds(dt * (_TPC * 1024), _TPC * 1024)],
            out_hbm.at[pl.ds(off, _TPC * 1024)], sem_s)

    def wait_idx():
      pltpu.make_async_copy(
          xt_hbm.at[pl.ds(0, _CHUNK)], idx_bufs[0], sem_i).wait()

    def wait_gather():
      pltpu.make_async_copy(
          table_hbm.at[idx_bufs[0]], row_bufs[0], sem_g).wait()

    def wait_store():
      for dt in range(_DIM // 8):
        pltpu.make_async_copy(
            tr_bufs[0].at[pl.ds(0, _TPC * 1024)],
            out_hbm.at[pl.ds(0, _TPC * 1024)], sem_s).wait()

    def transpose_unit(p):
      rows = row_bufs[p]
      tr = tr_bufs[p]

      @plsc.parallel_loop(0, _DIM, 1, unroll=2)
      def _(d):
        colv = (lane + d) & (_DIM - 1)
        destv = ((colv >> 3) * (_TPC * 1024) + (colv & 7) * 128) + lane
        for bl in range(_TPC):
          for g in range(8):
            rowv = lane + (bl * 128 + 16 * g)
            v = plsc.load_gather(rows, [rowv, colv])
            plsc.store_scatter(tr, [destv + (bl * 1024 + 16 * g)], v)

    issue_idx(0, 0)
    issue_idx(1, 1)
    wait_idx()
    issue_gather(0)

    def round_body(r, carry):
      for p in range(2):
        i = 2 * r + p
        @pl.when(i + 1 < _PER_W)
        def _():
          wait_idx()
          issue_gather(1 - p)

        wait_gather()

        @pl.when(i >= 2)
        def _():
          wait_store()

        transpose_unit(p)
        issue_store(i, p)

        @pl.when(i + 2 < _PER_W)
        def _():
          issue_idx(i + 2, p)
      return carry

    lax.fori_loop(0, _PER_W // 2, round_body, 0)
    wait_store()
    wait_store()

  return gather_kernel


_GATHER = _make_kernel()


def kernel(x, table):
  xt = jnp.transpose(x).reshape(-1).astype(jnp.int32)
  tp = jnp.pad(table, ((0, _VP - _V), (0, 0)))
  u4 = jnp.transpose(
      jnp.transpose(tp).reshape(_DIM // 8, 8, _TC7, 128), (0, 2, 1, 3))
  tconv = _CONV(u4.reshape(-1))
  u = _GATHER(xt, tconv.reshape(_VP, _DIM))
  u5 = u.reshape(_HIST, _DIM // 8, _BATCH // 128, 8, 128)
  v = jnp.transpose(u5, (0, 1, 3, 2, 4)).reshape(_HIST, _DIM, _BATCH)
  return jnp.transpose(v, (2, 0, 1))
